# baseline (speedup 1.0000x reference)
import jax
import jax.numpy as jnp
from jax import lax
from jax.experimental import pallas as pl
from jax.experimental.pallas import tpu as pltpu

DECAY = 0.999
UPDATE_RATE = 1.0
THRESH = 0.9
TEMP = 10.0
EPS = 1e-10


def _row_sum(x, rows, cols):
    t = x[:, :cols // 2] + x[:, cols // 2:]
    t3 = t.T.reshape(cols // 16, 8, rows)
    acc = t3[0]
    for m in range(1, cols // 16):
        acc = acc + t3[m]
    acc = acc[0:4] + acc[4:8]
    acc = acc[0:2] + acc[2:4]
    return acc[0:1] + acc[1:2]


def _rnn_kernel(inp_ref, pat_ref, mask_ref, out_ref, pats_ref, pat_scratch):
    t = pl.program_id(1)
    BP, H = pat_scratch.shape
    Bc = inp_ref.shape[1]
    P = BP // Bc

    @pl.when(t == 0)
    def _():
        pat_scratch[...] = pat_ref[...]

    pat2d = pat_scratch[...]
    h = inp_ref[0]

    raw_all = lax.dot_general(
        h, pat2d, (((1,), (1,)), ((), ())),
        preferred_element_type=jnp.float32)
    raw = jnp.concatenate(
        [raw_all[b:b + 1, b * P:(b + 1) * P] for b in range(Bc)], axis=0)

    mx = jnp.max(raw, axis=1, keepdims=True)
    masked = jnp.where(raw >= THRESH * mx, raw, 0.0)
    z = masked / mx * TEMP
    z = z - jnp.max(z, axis=1, keepdims=True)
    e = jnp.exp(z)
    den = _row_sum(e, Bc, P).T
    resp = e / den

    resp_blk = jnp.concatenate([resp] * Bc, axis=1) * mask_ref[...]
    new_h = lax.dot_general(
        resp_blk, pat2d, (((1,), (0,)), ((), ())),
        preferred_element_type=jnp.float32)

    up = resp[:, :, None] * h[:, None, :]
    newp = DECAY * pat2d + UPDATE_RATE * up.reshape(BP, H)
    sq = newp * newp
    inv = 1.0 / (jnp.sqrt(_row_sum(sq, BP, H)) + EPS)
    new_pat = newp * inv.T

    pat_scratch[...] = new_pat
    out_ref[0] = new_h
    pats_ref[0] = new_pat


def kernel(input, pat):
    T, B, H = input.shape
    _, P, _ = pat.shape
    BC = 2
    Bc = B // BC

    pat2d = pat.reshape(B * P, H)
    lane = lax.broadcasted_iota(jnp.int32, (Bc, Bc * P), 1)
    row = lax.broadcasted_iota(jnp.int32, (Bc, Bc * P), 0)
    mask = jnp.where(lane // P == row, 1.0, 0.0).astype(jnp.float32)

    out, all_pats = pl.pallas_call(
        _rnn_kernel,
        grid=(BC, T),
        in_specs=[
            pl.BlockSpec((1, Bc, H), lambda i, t: (t, i, 0)),
            pl.BlockSpec((Bc * P, H), lambda i, t: (i, 0)),
            pl.BlockSpec((Bc, Bc * P), lambda i, t: (0, 0)),
        ],
        out_specs=[
            pl.BlockSpec((1, Bc, H), lambda i, t: (t, i, 0)),
            pl.BlockSpec((1, Bc * P, H), lambda i, t: (t, i, 0)),
        ],
        out_shape=[
            jax.ShapeDtypeStruct((T, B, H), input.dtype),
            jax.ShapeDtypeStruct((T, B * P, H), input.dtype),
        ],
        scratch_shapes=[pltpu.VMEM((Bc * P, H), jnp.float32)],
        compiler_params=pltpu.CompilerParams(
            dimension_semantics=("arbitrary", "arbitrary"),
        ),
        name="pat_softmax_rnn",
    )(input, pat2d, mask)
    return out, all_pats.reshape(T, B, P, H)


# speedup vs baseline: 1.1951x; 1.1951x over previous
import jax
import jax.numpy as jnp
from jax import lax
from jax.experimental import pallas as pl
from jax.experimental.pallas import tpu as pltpu

DECAY = 0.999
UPDATE_RATE = 1.0
THRESH = 0.9
TEMP = 10.0
EPS = 1e-10


def _row_sum(x, rows, cols):
    t = x[:, :cols // 2] + x[:, cols // 2:]
    t3 = t.T.reshape(cols // 16, 8, rows)
    acc = t3[0]
    for m in range(1, cols // 16):
        acc = acc + t3[m]
    acc = acc[0:4] + acc[4:8]
    acc = acc[0:2] + acc[2:4]
    return acc[0:1] + acc[1:2]


def _rnn_kernel(inp_ref, pat_ref, mask_ref, out_ref, pats_ref, pat_scratch):
    t = pl.program_id(1)
    BP, H = pat_scratch.shape
    Bc = inp_ref.shape[1]
    P = BP // Bc

    @pl.when(t == 0)
    def _():
        pat_scratch[...] = pat_ref[...]

    pat2d = pat_scratch[...]
    h = inp_ref[0]

    raw_all = lax.dot_general(
        h, pat2d, (((1,), (1,)), ((), ())),
        preferred_element_type=jnp.float32)
    raw = jnp.concatenate(
        [raw_all[b:b + 1, b * P:(b + 1) * P] for b in range(Bc)], axis=0)

    mx = jnp.max(raw, axis=1, keepdims=True)
    masked = jnp.where(raw >= THRESH * mx, raw, 0.0)
    z = masked / mx * TEMP
    z = z - jnp.max(z, axis=1, keepdims=True)
    e = jnp.exp(z)
    den = _row_sum(e, Bc, P).T
    resp = e / den

    resp_blk = jnp.concatenate([resp] * Bc, axis=1) * mask_ref[...]
    new_h = lax.dot_general(
        resp_blk, pat2d, (((1,), (0,)), ((), ())),
        preferred_element_type=jnp.float32)

    up = resp[:, :, None] * h[:, None, :]
    newp = DECAY * pat2d + UPDATE_RATE * up.reshape(BP, H)
    sq = newp * newp
    inv = 1.0 / (jnp.sqrt(_row_sum(sq, BP, H)) + EPS)
    new_pat = newp * inv.T

    pat_scratch[...] = new_pat
    out_ref[0] = new_h
    pats_ref[0] = new_pat


def kernel(input, pat):
    T, B, H = input.shape
    _, P, _ = pat.shape
    BC = 1
    Bc = B // BC

    pat2d = pat.reshape(B * P, H)
    lane = lax.broadcasted_iota(jnp.int32, (Bc, Bc * P), 1)
    row = lax.broadcasted_iota(jnp.int32, (Bc, Bc * P), 0)
    mask = jnp.where(lane // P == row, 1.0, 0.0).astype(jnp.float32)

    out, all_pats = pl.pallas_call(
        _rnn_kernel,
        grid=(BC, T),
        in_specs=[
            pl.BlockSpec((1, Bc, H), lambda i, t: (t, i, 0)),
            pl.BlockSpec((Bc * P, H), lambda i, t: (i, 0)),
            pl.BlockSpec((Bc, Bc * P), lambda i, t: (0, 0)),
        ],
        out_specs=[
            pl.BlockSpec((1, Bc, H), lambda i, t: (t, i, 0)),
            pl.BlockSpec((1, Bc * P, H), lambda i, t: (t, i, 0)),
        ],
        out_shape=[
            jax.ShapeDtypeStruct((T, B, H), input.dtype),
            jax.ShapeDtypeStruct((T, B * P, H), input.dtype),
        ],
        scratch_shapes=[pltpu.VMEM((Bc * P, H), jnp.float32)],
        compiler_params=pltpu.CompilerParams(
            dimension_semantics=("arbitrary", "arbitrary"),
        ),
        name="pat_softmax_rnn",
    )(input, pat2d, mask)
    return out, all_pats.reshape(T, B, P, H)


# speedup vs baseline: 1.1963x; 1.0011x over previous
import jax
import jax.numpy as jnp
from jax import lax
from jax.experimental import pallas as pl
from jax.experimental.pallas import tpu as pltpu

DECAY = 0.999
UPDATE_RATE = 1.0
THRESH = 0.9
TEMP = 10.0
EPS = 1e-10


def _row_sum(x, rows, cols):
    t = x[:, :cols // 2] + x[:, cols // 2:]
    t3 = t.T.reshape(cols // 16, 8, rows)
    acc = t3[0]
    for m in range(1, cols // 16):
        acc = acc + t3[m]
    acc = acc[0:4] + acc[4:8]
    acc = acc[0:2] + acc[2:4]
    return acc[0:1] + acc[1:2]


def _rnn_kernel(inp_ref, pat_ref, mask_ref, out_ref, pats_ref, pat_scratch):
    t = pl.program_id(0)
    BP, H = pat_scratch.shape
    Bc = inp_ref.shape[1]
    P = BP // Bc

    @pl.when(t == 0)
    def _():
        pat_scratch[...] = pat_ref[...]

    pat2d = pat_scratch[...]
    h = inp_ref[0]

    raw_all = lax.dot_general(
        h, pat2d, (((1,), (1,)), ((), ())),
        preferred_element_type=jnp.float32)
    raw = jnp.concatenate(
        [raw_all[b:b + 1, b * P:(b + 1) * P] for b in range(Bc)], axis=0)

    mx = jnp.max(raw, axis=1, keepdims=True)
    masked = jnp.where(raw >= THRESH * mx, raw, 0.0)
    z = masked / mx * TEMP
    z = z - jnp.max(z, axis=1, keepdims=True)
    e = jnp.exp(z)
    den = _row_sum(e, Bc, P).T
    resp = e / den

    resp_blk = jnp.concatenate([resp] * Bc, axis=1) * mask_ref[...]
    new_h = lax.dot_general(
        resp_blk, pat2d, (((1,), (0,)), ((), ())),
        preferred_element_type=jnp.float32)

    up = resp[:, :, None] * h[:, None, :]
    newp = DECAY * pat2d + UPDATE_RATE * up.reshape(BP, H)
    sq = newp * newp
    inv = 1.0 / (jnp.sqrt(_row_sum(sq, BP, H)) + EPS)
    new_pat = newp * inv.T

    pat_scratch[...] = new_pat
    out_ref[0] = new_h
    pats_ref[0] = new_pat


def kernel(input, pat):
    T, B, H = input.shape
    _, P, _ = pat.shape
    BC = 1
    Bc = B // BC

    pat2d = pat.reshape(B * P, H)
    lane = lax.broadcasted_iota(jnp.int32, (Bc, Bc * P), 1)
    row = lax.broadcasted_iota(jnp.int32, (Bc, Bc * P), 0)
    mask = jnp.where(lane // P == row, 1.0, 0.0).astype(jnp.float32)

    out, all_pats = pl.pallas_call(
        _rnn_kernel,
        grid=(T,),
        in_specs=[
            pl.BlockSpec((1, Bc, H), lambda t: (t, 0, 0)),
            pl.BlockSpec((Bc * P, H), lambda t: (0, 0)),
            pl.BlockSpec((Bc, Bc * P), lambda t: (0, 0)),
        ],
        out_specs=[
            pl.BlockSpec((1, Bc, H), lambda t: (t, 0, 0)),
            pl.BlockSpec((1, Bc * P, H), lambda t: (t, 0, 0)),
        ],
        out_shape=[
            jax.ShapeDtypeStruct((T, B, H), input.dtype),
            jax.ShapeDtypeStruct((T, B * P, H), input.dtype),
        ],
        scratch_shapes=[pltpu.VMEM((Bc * P, H), jnp.float32)],
        compiler_params=pltpu.CompilerParams(
            dimension_semantics=("arbitrary",),
        ),
        name="pat_softmax_rnn",
    )(input, pat2d, mask)
    return out, all_pats.reshape(T, B, P, H)
